# baseline (speedup 1.0000x reference)
import functools

import jax
import jax.numpy as jnp
from jax import lax
from jax.experimental import pallas as pl
from jax.experimental.pallas import tpu as pltpu
from jax.experimental.pallas import tpu_sc as plsc

B = 16384
L = 200
DIM = 64
NC, NS = 2, 16
NW = NC * NS
RPW = B // NW
GR = 16
NGRP = RPW // GR
WPAD = 208
CH = 40
NCH = L // CH


def _sc_body(x_hbm, len_hbm, emb_hbm, wt_hbm, out_hbm,
             idx_v, emb_v, w_v, len_v, out_v,
             isem, esem0, esem1, wsem0, wsem1, osem):
  esem = (esem0, esem1)
  wsem = (wsem0, wsem1)
  wid = lax.axis_index("s") * NC + lax.axis_index("c")
  base = wid * RPW

  pltpu.sync_copy(len_hbm.at[pl.ds(base, RPW)], len_v)
  pltpu.async_copy(x_hbm.at[pl.ds(base * L, GR * L)], idx_v.at[0], isem)

  zvec = jnp.zeros((16,), jnp.float32)

  @pl.loop(0, WPAD)
  def _zinit(t):
    for ebi in range(2):
      for k in range(4):
        emb_v[ebi, t, pl.ds(16 * k, 16)] = zvec

  def fetch(db, j, eb, lnv):
    nch = (lnv[j] + (CH - 1)) // CH
    for c in range(NCH):
      @pl.when(c < nch)
      def _(c=c):
        pltpu.async_copy(emb_hbm.at[idx_v.at[db, pl.ds(j * L + c * CH, CH)]],
                         emb_v.at[eb, pl.ds(c * CH, CH)], esem[eb])
        pltpu.async_copy(wt_hbm.at[idx_v.at[db, pl.ds(j * L + c * CH, CH)]],
                         w_v.at[eb, pl.ds(c * CH, CH)], wsem[eb])

  def wait_fetch(db, j, eb, lnv):
    nch = (lnv[j] + (CH - 1)) // CH
    for c in range(NCH):
      @pl.when(c < nch)
      def _(c=c):
        pltpu.make_async_copy(emb_hbm.at[idx_v.at[db, pl.ds(j * L + c * CH, CH)]],
                              emb_v.at[eb, pl.ds(c * CH, CH)], esem[eb]).wait()
        pltpu.make_async_copy(wt_hbm.at[idx_v.at[db, pl.ds(j * L + c * CH, CH)]],
                              w_v.at[eb, pl.ds(c * CH, CH)], wsem[eb]).wait()

  @pl.loop(0, NGRP)
  def _group(g):
    db = lax.rem(g, 2)
    rowbase = base + g * GR
    pltpu.make_async_copy(x_hbm.at[pl.ds(rowbase * L, GR * L)], idx_v.at[db],
                          isem).wait()

    @pl.when(g + 1 < NGRP)
    def _():
      pltpu.async_copy(x_hbm.at[pl.ds((rowbase + GR) * L, GR * L)],
                       idx_v.at[1 - db], isem)

    lnv = len_v[pl.ds(g * GR, GR)]
    fetch(db, 0, 0, lnv)
    for j in range(GR):
      eb = j % 2
      if j + 1 < GR:
        fetch(db, j + 1, 1 - eb, lnv)
      wait_fetch(db, j, eb, lnv)

      ln = lnv[j]
      nmg = (ln + 15) // 16
      zero = jnp.zeros((16,), jnp.float32)

      def grp(m, acc, eb=eb, ln=ln):
        wg = w_v[eb, pl.ds(m * 16, 16)]
        pos = lax.iota(jnp.int32, 16) + m * 16
        wgm = jnp.where(pos < ln, wg, 0.0)
        for t in range(16):
          wt = wgm[t]
          acc = tuple(
              acc[k] + wt * emb_v[eb, m * 16 + t, pl.ds(16 * k, 16)]
              for k in range(4))
        return acc

      acc = lax.fori_loop(0, nmg, grp, (zero, zero, zero, zero))
      for k in range(4):
        out_v[db, j, pl.ds(16 * k, 16)] = acc[k]

    @pl.when(g > 0)
    def _():
      pltpu.make_async_copy(out_v.at[1 - db],
                            out_hbm.at[pl.ds(rowbase - GR, GR)], osem).wait()

    pltpu.async_copy(out_v.at[db], out_hbm.at[pl.ds(rowbase, GR)], osem)

  pltpu.make_async_copy(out_v.at[(NGRP - 1) % 2],
                        out_hbm.at[pl.ds(base + (NGRP - 1) * GR, GR)],
                        osem).wait()


VOCAB = 1000000
VB = 400
NVB = VOCAB // VB


def _sc_transpose_body(soa_hbm, rm_hbm, s0_v, s1_v, o0_v, o1_v,
                       tsem0, tsem1, osem0, osem1):
  wid = lax.axis_index("s") * NC + lax.axis_index("c")
  tsem = (tsem0, tsem1)
  osem = (osem0, osem1)
  strips = (s0_v, s1_v)
  outs = (o0_v, o1_v)

  def issue(i, hb):
    for d in range(DIM):
      pltpu.async_copy(soa_hbm.at[pl.ds(d * VOCAB + i * VB, VB)],
                       strips[hb].at[d], tsem[hb])

  def wait_strips(i, hb):
    for d in range(DIM):
      pltpu.make_async_copy(soa_hbm.at[pl.ds(d * VOCAB + i * VB, VB)],
                            strips[hb].at[d], tsem[hb]).wait()

  def transpose_block(i, hb):
    wait_strips(i, hb)

    @pl.loop(0, VB)
    def _row(r):
      rsplat = jnp.full((16,), 0, jnp.int32) + r
      for g in range(4):
        dvec = lax.iota(jnp.int32, 16) + (16 * g)
        vals = plsc.load_gather(strips[hb], [dvec, rsplat])
        outs[hb][pl.ds(r * DIM + 16 * g, 16)] = vals

    pltpu.async_copy(outs[hb], rm_hbm.at[pl.ds(i * VB * DIM, VB * DIM)],
                     osem[hb])

  def wait_out(i, hb):
    pltpu.make_async_copy(outs[hb], rm_hbm.at[pl.ds(i * VB * DIM, VB * DIM)],
                          osem[hb]).wait()

  issue(wid, 0)

  @pl.loop(wid, NVB, step=2 * NW)
  def _blk(i):
    i2 = i + NW

    @pl.when(i2 < NVB)
    def _():
      issue(i2, 1)

    @pl.when(i > wid)
    def _():
      wait_out(i - 2 * NW, 0)

    transpose_block(i, 0)

    @pl.when(i2 < NVB)
    def _():
      @pl.when(i2 + NW < NVB)
      def _():
        issue(i2 + NW, 0)

      @pl.when(i > wid)
      def _():
        wait_out(i - NW, 1)

      transpose_block(i2, 1)

  last = wid + ((NVB - 1 - wid) // NW) * NW
  @pl.when(last >= 0)
  def _():
    @pl.when((((last - wid) // NW) % 2) == 0)
    def _():
      wait_out(last, 0)
      @pl.when(last - NW >= 0)
      def _():
        wait_out(last - NW, 1)

    @pl.when((((last - wid) // NW) % 2) == 1)
    def _():
      wait_out(last, 1)
      @pl.when(last - NW >= 0)
      def _():
        wait_out(last - NW, 0)


def _sc_transpose(soa_flat):
  mesh = plsc.VectorSubcoreMesh(core_axis_name="c", subcore_axis_name="s",
                                num_cores=NC, num_subcores=NS)
  f = pl.kernel(
      _sc_transpose_body,
      out_type=jax.ShapeDtypeStruct((VOCAB * DIM,), jnp.float32),
      mesh=mesh,
      compiler_params=pltpu.CompilerParams(use_tc_tiling_on_sc=False,
                                           needs_layout_passes=False),
      scratch_types=[
          pltpu.VMEM((DIM, VB), jnp.float32),
          pltpu.VMEM((DIM, VB), jnp.float32),
          pltpu.VMEM((VB * DIM,), jnp.float32),
          pltpu.VMEM((VB * DIM,), jnp.float32),
          pltpu.SemaphoreType.DMA,
          pltpu.SemaphoreType.DMA,
          pltpu.SemaphoreType.DMA,
          pltpu.SemaphoreType.DMA,
      ],
  )
  return f(soa_flat)


def _sc_weighted_sums(x, lengths, emb_table, wt_flat):
  mesh = plsc.VectorSubcoreMesh(core_axis_name="c", subcore_axis_name="s",
                                num_cores=NC, num_subcores=NS)
  f = pl.kernel(
      _sc_body,
      out_type=jax.ShapeDtypeStruct((B, DIM), jnp.float32),
      mesh=mesh,
      compiler_params=pltpu.CompilerParams(use_tc_tiling_on_sc=False),
      scratch_types=[
          pltpu.VMEM((2, GR * L), jnp.int32),
          pltpu.VMEM((2, WPAD, DIM), jnp.float32),
          pltpu.VMEM((2, WPAD), jnp.float32),
          pltpu.VMEM((RPW,), jnp.int32),
          pltpu.VMEM((2, GR, DIM), jnp.float32),
          pltpu.SemaphoreType.DMA,
          pltpu.SemaphoreType.DMA,
          pltpu.SemaphoreType.DMA,
          pltpu.SemaphoreType.DMA,
          pltpu.SemaphoreType.DMA,
          pltpu.SemaphoreType.DMA,
      ],
  )
  return f(x.reshape(-1), lengths, emb_table, wt_flat)


def _tc_body(s_ref, l_ref, w_ref, b_ref, o_ref):
  avg = s_ref[...] / l_ref[...].astype(jnp.float32)
  o_ref[...] = (
      jnp.dot(avg, w_ref[...], preferred_element_type=jnp.float32)
      + b_ref[...])


def _tc_project(summed, lengths, W_lin, b_lin):
  BLK = 2048
  return pl.pallas_call(
      _tc_body,
      grid=(B // BLK,),
      in_specs=[
          pl.BlockSpec((BLK, DIM), lambda i: (i, 0)),
          pl.BlockSpec((BLK, 1), lambda i: (i, 0)),
          pl.BlockSpec((DIM, DIM), lambda i: (0, 0)),
          pl.BlockSpec((1, DIM), lambda i: (0, 0)),
      ],
      out_specs=pl.BlockSpec((BLK, DIM), lambda i: (i, 0)),
      out_shape=jax.ShapeDtypeStruct((B, DIM), jnp.float32),
  )(summed, lengths.reshape(B, 1), W_lin, b_lin.reshape(1, DIM))


@jax.jit
def kernel(x, lengths, emb_table, weight_table, W_lin, b_lin):
  wt_flat = weight_table.reshape(-1)
  emb_soa = emb_table.T.reshape(-1)
  emb_rm = _sc_transpose(emb_soa).reshape(VOCAB, DIM)
  summed = _sc_weighted_sums(x, lengths, emb_rm, wt_flat)
  return _tc_project(summed, lengths, W_lin, b_lin)


# speedup vs baseline: 4.5492x; 4.5492x over previous
import functools

import jax
import jax.numpy as jnp
from jax import lax
from jax.experimental import pallas as pl
from jax.experimental.pallas import tpu as pltpu
from jax.experimental.pallas import tpu_sc as plsc

B = 16384
L = 200
DIM = 64
NC, NS = 2, 16
NW = NC * NS
RPW = B // NW
GR = 8
NGRP = RPW // GR
WPAD = 208
CS = (64, 64, 64, 8)
CO = (0, 64, 128, 192)


def _sc_body(x_hbm, len_hbm, emb_hbm, wt_hbm, out_hbm,
             idx_v, emb_v, w_v, len_v, out_v,
             isem, esem0, esem1, wsem0, wsem1, osem):
  esem = (esem0, esem1)
  wsem = (wsem0, wsem1)
  wid = lax.axis_index("s") * NC + lax.axis_index("c")
  base = wid * RPW

  pltpu.sync_copy(len_hbm.at[pl.ds(base, RPW)], len_v)
  pltpu.async_copy(x_hbm.at[pl.ds(base * L, GR * L)], idx_v.at[0], isem)

  zvec = jnp.zeros((16,), jnp.float32)

  @pl.loop(0, WPAD)
  def _zinit(t):
    for ebi in range(2):
      for k in range(4):
        emb_v[ebi, t, pl.ds(16 * k, 16)] = zvec

  def fetch(db, j, eb, lnv):
    nch = (lnv[j] + 63) // 64
    for c in range(4):
      @pl.when(c < nch)
      def _(c=c):
        pltpu.async_copy(emb_hbm.at[idx_v.at[db, pl.ds(j * L + CO[c], CS[c])]],
                         emb_v.at[eb, pl.ds(CO[c], CS[c])], esem[eb])
        pltpu.async_copy(wt_hbm.at[idx_v.at[db, pl.ds(j * L + CO[c], CS[c])]],
                         w_v.at[eb, pl.ds(CO[c], CS[c])], wsem[eb])

  def wait_chunk(db, j, eb, c):
    pltpu.make_async_copy(emb_hbm.at[idx_v.at[db, pl.ds(j * L + CO[c], CS[c])]],
                          emb_v.at[eb, pl.ds(CO[c], CS[c])], esem[eb]).wait()
    pltpu.make_async_copy(wt_hbm.at[idx_v.at[db, pl.ds(j * L + CO[c], CS[c])]],
                          w_v.at[eb, pl.ds(CO[c], CS[c])], wsem[eb]).wait()

  @pl.loop(0, NGRP)
  def _group(g):
    db = lax.rem(g, 2)
    rowbase = base + g * GR
    pltpu.make_async_copy(x_hbm.at[pl.ds(rowbase * L, GR * L)], idx_v.at[db],
                          isem).wait()

    @pl.when(g + 1 < NGRP)
    def _():
      pltpu.async_copy(x_hbm.at[pl.ds((rowbase + GR) * L, GR * L)],
                       idx_v.at[1 - db], isem)

    lnv = len_v[pl.ds(g * GR, GR)]
    fetch(db, 0, 0, lnv)
    for j in range(GR):
      eb = j % 2
      if j + 1 < GR:
        fetch(db, j + 1, 1 - eb, lnv)

      ln = lnv[j]
      nch = (ln + 63) // 64
      nmg = (ln + 15) // 16
      zero = jnp.zeros((16,), jnp.float32)

      def grp(m, acc, eb=eb, ln=ln):
        wg = w_v[eb, pl.ds(m * 16, 16)]
        pos = lax.iota(jnp.int32, 16) + m * 16
        wgm = jnp.where(pos < ln, wg, 0.0)
        for t in range(16):
          wt = wgm[t]
          acc = tuple(
              acc[k] + wt * emb_v[eb, m * 16 + t, pl.ds(16 * k, 16)]
              for k in range(4))
        return acc

      acc = (zero, zero, zero, zero)
      for c in range(4):
        @pl.when(c < nch)
        def _(c=c):
          wait_chunk(db, j, eb, c)
        acc = lax.fori_loop(4 * c, jnp.minimum(4 * (c + 1), nmg), grp, acc)
      for k in range(4):
        out_v[db, j, pl.ds(16 * k, 16)] = acc[k]

    @pl.when(g > 0)
    def _():
      pltpu.make_async_copy(out_v.at[1 - db],
                            out_hbm.at[pl.ds(rowbase - GR, GR)], osem).wait()

    pltpu.async_copy(out_v.at[db], out_hbm.at[pl.ds(rowbase, GR)], osem)

  pltpu.make_async_copy(out_v.at[(NGRP - 1) % 2],
                        out_hbm.at[pl.ds(base + (NGRP - 1) * GR, GR)],
                        osem).wait()


def _sc_weighted_sums(x, lengths, emb_table, wt_flat):
  mesh = plsc.VectorSubcoreMesh(core_axis_name="c", subcore_axis_name="s",
                                num_cores=NC, num_subcores=NS)
  f = pl.kernel(
      _sc_body,
      out_type=jax.ShapeDtypeStruct((B, DIM), jnp.float32),
      mesh=mesh,
      compiler_params=pltpu.CompilerParams(use_tc_tiling_on_sc=False),
      scratch_types=[
          pltpu.VMEM((2, GR * L), jnp.int32),
          pltpu.VMEM((2, WPAD, DIM), jnp.float32),
          pltpu.VMEM((2, WPAD), jnp.float32),
          pltpu.VMEM((RPW,), jnp.int32),
          pltpu.VMEM((2, GR, DIM), jnp.float32),
          pltpu.SemaphoreType.DMA,
          pltpu.SemaphoreType.DMA,
          pltpu.SemaphoreType.DMA,
          pltpu.SemaphoreType.DMA,
          pltpu.SemaphoreType.DMA,
          pltpu.SemaphoreType.DMA,
      ],
  )
  return f(x.reshape(-1), lengths, emb_table, wt_flat)


def _tc_body(s_ref, l_ref, w_ref, b_ref, o_ref):
  avg = s_ref[...] / l_ref[...].astype(jnp.float32)
  o_ref[...] = (
      jnp.dot(avg, w_ref[...], preferred_element_type=jnp.float32)
      + b_ref[...])


def _tc_project(summed, lengths, W_lin, b_lin):
  BLK = 2048
  return pl.pallas_call(
      _tc_body,
      grid=(B // BLK,),
      in_specs=[
          pl.BlockSpec((BLK, DIM), lambda i: (i, 0)),
          pl.BlockSpec((BLK, 1), lambda i: (i, 0)),
          pl.BlockSpec((DIM, DIM), lambda i: (0, 0)),
          pl.BlockSpec((1, DIM), lambda i: (0, 0)),
      ],
      out_specs=pl.BlockSpec((BLK, DIM), lambda i: (i, 0)),
      out_shape=jax.ShapeDtypeStruct((B, DIM), jnp.float32),
  )(summed, lengths.reshape(B, 1), W_lin, b_lin.reshape(1, DIM))


@jax.jit
def kernel(x, lengths, emb_table, weight_table, W_lin, b_lin):
  wt_flat = weight_table.reshape(-1)
  summed = _sc_weighted_sums(x, lengths, emb_table, wt_flat)
  return _tc_project(summed, lengths, W_lin, b_lin)


# speedup vs baseline: 5.9939x; 1.3176x over previous
import functools

import jax
import jax.numpy as jnp
from jax import lax
from jax.experimental import pallas as pl
from jax.experimental.pallas import tpu as pltpu
from jax.experimental.pallas import tpu_sc as plsc

B = 16384
L = 200
DIM = 64
NC, NS = 2, 16
NW = NC * NS
RPW = B // NW
GR = 16
NGRP = RPW // GR
WPAD = 208
CS = (64, 64, 64, 8)
CO = (0, 64, 128, 192)


def _sc_body(x_hbm, len_hbm, emb_hbm, wt_hbm, out_hbm,
             idx_v, emb_v, w_v, len_v, out_v,
             isem, esem0, esem1, wsem0, wsem1, osem):
  esem = (esem0, esem1)
  wsem = (wsem0, wsem1)
  wid = lax.axis_index("s") * NC + lax.axis_index("c")
  base = wid * RPW

  pltpu.sync_copy(len_hbm.at[pl.ds(base, RPW)], len_v)
  pltpu.async_copy(x_hbm.at[pl.ds(base * L, GR * L)], idx_v.at[0], isem)

  zvec = jnp.zeros((16,), jnp.float32)

  @pl.loop(0, WPAD)
  def _zinit(t):
    for ebi in range(2):
      for k in range(4):
        emb_v[ebi, t, pl.ds(16 * k, 16)] = zvec

  def fetch(db, j, eb, lnv):
    nch = (lnv[j] + 63) // 64
    for c in range(4):
      @pl.when(c < nch)
      def _(c=c):
        pltpu.async_copy(emb_hbm.at[idx_v.at[db, pl.ds(j * L + CO[c], CS[c])]],
                         emb_v.at[eb, pl.ds(CO[c], CS[c])], esem[eb])
        pltpu.async_copy(wt_hbm.at[idx_v.at[db, pl.ds(j * L + CO[c], CS[c])]],
                         w_v.at[eb, pl.ds(CO[c], CS[c])], wsem[eb])

  def wait_chunk(db, j, eb, c):
    pltpu.make_async_copy(emb_hbm.at[idx_v.at[db, pl.ds(j * L + CO[c], CS[c])]],
                          emb_v.at[eb, pl.ds(CO[c], CS[c])], esem[eb]).wait()
    pltpu.make_async_copy(wt_hbm.at[idx_v.at[db, pl.ds(j * L + CO[c], CS[c])]],
                          w_v.at[eb, pl.ds(CO[c], CS[c])], wsem[eb]).wait()

  @pl.loop(0, NGRP)
  def _group(g):
    db = lax.rem(g, 2)
    rowbase = base + g * GR
    pltpu.make_async_copy(x_hbm.at[pl.ds(rowbase * L, GR * L)], idx_v.at[db],
                          isem).wait()

    @pl.when(g + 1 < NGRP)
    def _():
      pltpu.async_copy(x_hbm.at[pl.ds((rowbase + GR) * L, GR * L)],
                       idx_v.at[1 - db], isem)

    lnv = len_v[pl.ds(g * GR, GR)]
    fetch(db, 0, 0, lnv)
    for j in range(GR):
      eb = j % 2
      if j + 1 < GR:
        fetch(db, j + 1, 1 - eb, lnv)

      ln = lnv[j]
      nch = (ln + 63) // 64
      nmg = (ln + 15) // 16
      zero = jnp.zeros((16,), jnp.float32)

      def grp(m, acc, eb=eb, ln=ln):
        wg = w_v[eb, pl.ds(m * 16, 16)]
        pos = lax.iota(jnp.int32, 16) + m * 16
        wgm = jnp.where(pos < ln, wg, 0.0)
        for t in range(16):
          wt = wgm[t]
          acc = tuple(
              acc[k] + wt * emb_v[eb, m * 16 + t, pl.ds(16 * k, 16)]
              for k in range(4))
        return acc

      for c in range(4):
        @pl.when(c < nch)
        def _(c=c):
          wait_chunk(db, j, eb, c)
      acc = lax.fori_loop(0, nmg, grp, (zero, zero, zero, zero))
      for k in range(4):
        out_v[db, j, pl.ds(16 * k, 16)] = acc[k]

    @pl.when(g > 0)
    def _():
      pltpu.make_async_copy(out_v.at[1 - db],
                            out_hbm.at[pl.ds(rowbase - GR, GR)], osem).wait()

    pltpu.async_copy(out_v.at[db], out_hbm.at[pl.ds(rowbase, GR)], osem)

  pltpu.make_async_copy(out_v.at[(NGRP - 1) % 2],
                        out_hbm.at[pl.ds(base + (NGRP - 1) * GR, GR)],
                        osem).wait()


def _sc_weighted_sums(x, lengths, emb_table, wt_flat):
  mesh = plsc.VectorSubcoreMesh(core_axis_name="c", subcore_axis_name="s",
                                num_cores=NC, num_subcores=NS)
  f = pl.kernel(
      _sc_body,
      out_type=jax.ShapeDtypeStruct((B, DIM), jnp.float32),
      mesh=mesh,
      compiler_params=pltpu.CompilerParams(use_tc_tiling_on_sc=False),
      scratch_types=[
          pltpu.VMEM((2, GR * L), jnp.int32),
          pltpu.VMEM((2, WPAD, DIM), jnp.float32),
          pltpu.VMEM((2, WPAD), jnp.float32),
          pltpu.VMEM((RPW,), jnp.int32),
          pltpu.VMEM((2, GR, DIM), jnp.float32),
          pltpu.SemaphoreType.DMA,
          pltpu.SemaphoreType.DMA,
          pltpu.SemaphoreType.DMA,
          pltpu.SemaphoreType.DMA,
          pltpu.SemaphoreType.DMA,
          pltpu.SemaphoreType.DMA,
      ],
  )
  return f(x.reshape(-1), lengths, emb_table, wt_flat)


def _tc_body(s_ref, l_ref, w_ref, b_ref, o_ref):
  avg = s_ref[...] / l_ref[...].astype(jnp.float32)
  o_ref[...] = (
      jnp.dot(avg, w_ref[...], preferred_element_type=jnp.float32)
      + b_ref[...])


def _tc_project(summed, lengths, W_lin, b_lin):
  BLK = 2048
  return pl.pallas_call(
      _tc_body,
      grid=(B // BLK,),
      in_specs=[
          pl.BlockSpec((BLK, DIM), lambda i: (i, 0)),
          pl.BlockSpec((BLK, 1), lambda i: (i, 0)),
          pl.BlockSpec((DIM, DIM), lambda i: (0, 0)),
          pl.BlockSpec((1, DIM), lambda i: (0, 0)),
      ],
      out_specs=pl.BlockSpec((BLK, DIM), lambda i: (i, 0)),
      out_shape=jax.ShapeDtypeStruct((B, DIM), jnp.float32),
  )(summed, lengths.reshape(B, 1), W_lin, b_lin.reshape(1, DIM))


@jax.jit
def kernel(x, lengths, emb_table, weight_table, W_lin, b_lin):
  wt_flat = weight_table.reshape(-1)
  summed = _sc_weighted_sums(x, lengths, emb_table, wt_flat)
  return _tc_project(summed, lengths, W_lin, b_lin)
